# Initial kernel scaffold; baseline (speedup 1.0000x reference)
import jax
import jax.numpy as jnp
from jax.experimental import pallas as pl


def kernel(x, edge_index, W1, b1, W2, b2, Wlin, blin):
    raise NotImplementedError("write your pallas kernel here")


# speedup vs baseline: 8.6279x; 8.6279x over previous
import functools

import jax
import jax.numpy as jnp
from jax import lax
from jax.experimental import pallas as pl
from jax.experimental.pallas import tpu as pltpu
from jax.experimental.pallas import tpu_sc as plsc

N = 10000
E = 320000
FD = 128
NW = 32
RPT = 640
NACC = 16 * RPT
CH = 128
NCH = 79
EPW = NCH * CH
EPAD = NW * EPW
RBLK = 640

_mesh = plsc.VectorSubcoreMesh(core_axis_name="c", subcore_axis_name="s")

_f32 = jnp.float32
_i32 = jnp.int32


@functools.partial(
    pl.kernel,
    mesh=_mesh,
    out_type=(
        jax.ShapeDtypeStruct((EPAD,), _i32),
        jax.ShapeDtypeStruct((EPAD,), _i32),
        jax.ShapeDtypeStruct((2, NACC), _f32),
    ),
    scratch_types=[
        pltpu.VMEM((CH,), _i32),
        pltpu.VMEM((CH,), _i32),
        pltpu.VMEM((CH,), _f32),
        pltpu.VMEM((RPT,), _f32),
        pltpu.VMEM_SHARED((NACC,), _f32),
        pltpu.SemaphoreType.DMA,
    ],
)
def _prep(ei, srcf, dstf, degp, rv, cv, ov, bv, dacc, sem):
    del sem
    cid = lax.axis_index("c")
    sid = lax.axis_index("s")
    wid = sid * 2 + cid

    def zb(i, carry):
        bv[pl.ds(i * 16, 16)] = jnp.zeros((16,), _f32)
        return carry

    lax.fori_loop(0, RPT // 16, zb, 0)
    pltpu.sync_copy(bv, dacc.at[pl.ds(sid * RPT, RPT)])
    plsc.subcore_barrier()

    def chunk(k, carry):
        base = wid * EPW + k * CH

        @pl.when(base < E)
        def _real():
            pltpu.sync_copy(ei.at[0, pl.ds(base, CH)], rv)
            pltpu.sync_copy(ei.at[1, pl.ds(base, CH)], cv)
            for j in range(CH // 16):
                sl = pl.ds(j * 16, 16)
                r = rv[sl]
                c = cv[sl]
                m = r == c
                rz = N + (r & 7)
                rv[sl] = jnp.where(m, rz, r)
                ov[sl] = jnp.where(m, jnp.zeros((16,), _f32),
                                   jnp.ones((16,), _f32))
            pltpu.sync_copy(rv, srcf.at[pl.ds(base, CH)])
            pltpu.sync_copy(cv, dstf.at[pl.ds(base, CH)])
            pltpu.sync_copy(ov, dacc.at[rv], add=True)

        @pl.when(base >= E)
        def _pad():
            ii = lax.iota(_i32, 16)
            for j in range(CH // 16):
                sl = pl.ds(j * 16, 16)
                rv[sl] = N + (ii & 7)
                cv[sl] = N + (ii & 15)
            pltpu.sync_copy(rv, srcf.at[pl.ds(base, CH)])
            pltpu.sync_copy(cv, dstf.at[pl.ds(base, CH)])

        return carry

    lax.fori_loop(0, NCH, chunk, 0)
    plsc.subcore_barrier()

    pltpu.sync_copy(dacc.at[pl.ds(sid * RPT, RPT)], bv)
    pltpu.sync_copy(bv, degp.at[cid, pl.ds(sid * RPT, RPT)])


@functools.partial(
    pl.kernel,
    mesh=_mesh,
    out_type=jax.ShapeDtypeStruct((2, NACC, FD), _f32),
    scratch_types=[
        pltpu.VMEM((CH,), _i32),
        pltpu.VMEM((CH,), _i32),
        pltpu.VMEM((CH, FD), _f32),
        pltpu.VMEM_SHARED((NACC, FD), _f32),
        pltpu.SemaphoreType.DMA,
    ],
)
def _prop(tbl, src, dst, out, si, di, rows, acc, sem):
    cid = lax.axis_index("c")
    sid = lax.axis_index("s")
    wid = sid * 2 + cid

    def zrow(i, carry):
        for j in range(FD // 16):
            rows[i, pl.ds(j * 16, 16)] = jnp.zeros((16,), _f32)
        return carry

    lax.fori_loop(0, CH, zrow, 0)
    for q in range(RPT // CH):
        pltpu.sync_copy(rows, acc.at[pl.ds(sid * RPT + q * CH, CH)])
    plsc.subcore_barrier()

    def chunk(k, carry):
        base = wid * EPW + k * CH
        pltpu.sync_copy(src.at[pl.ds(base, CH)], si)
        pltpu.sync_copy(dst.at[pl.ds(base, CH)], di)
        pltpu.async_copy(tbl.at[si], rows, sem).wait()
        pltpu.sync_copy(rows, acc.at[di], add=True)
        return carry

    lax.fori_loop(0, NCH, chunk, 0)
    plsc.subcore_barrier()

    for q in range(RPT // CH):
        r0 = sid * RPT + q * CH
        pltpu.sync_copy(acc.at[pl.ds(r0, CH)], rows)
        pltpu.sync_copy(rows, out.at[cid, pl.ds(r0, CH)])


def _dis_body(degp_ref, dis_ref, dgi_ref):
    deg = degp_ref[0] + degp_ref[1]
    good = deg > 0.0
    safe = jnp.where(good, deg, 1.0)
    r = lax.rsqrt(safe)
    dis_ref[...] = jnp.where(good, r, 0.0)
    dgi_ref[...] = jnp.where(good, 1.0 / safe, 0.0)


def _dis(degp):
    d2 = degp.reshape(2, NACC // 128, 128)
    return pl.pallas_call(
        _dis_body,
        out_shape=(
            jax.ShapeDtypeStruct((NACC // 128, 128), _f32),
            jax.ShapeDtypeStruct((NACC // 128, 128), _f32),
        ),
    )(d2)


def _scale_body(x_ref, s_ref, o_ref):
    o_ref[...] = x_ref[...] * s_ref[...]


def _scale(x, s):
    return pl.pallas_call(
        _scale_body,
        grid=(NACC // RBLK,),
        in_specs=[
            pl.BlockSpec((RBLK, FD), lambda i: (i, 0)),
            pl.BlockSpec((RBLK, 1), lambda i: (i, 0)),
        ],
        out_specs=pl.BlockSpec((RBLK, FD), lambda i: (i, 0)),
        out_shape=jax.ShapeDtypeStruct((NACC, FD), _f32),
    )(x, s)


def _mid_body(p_ref, g_ref, o_ref):
    o_ref[...] = (p_ref[0] + p_ref[1]) * (-g_ref[...])


def _mid(p, g):
    return pl.pallas_call(
        _mid_body,
        grid=(NACC // RBLK,),
        in_specs=[
            pl.BlockSpec((2, RBLK, FD), lambda i: (0, i, 0)),
            pl.BlockSpec((RBLK, 1), lambda i: (i, 0)),
        ],
        out_specs=pl.BlockSpec((RBLK, FD), lambda i: (i, 0)),
        out_shape=jax.ShapeDtypeStruct((NACC, FD), _f32),
    )(p, g)


def _layer_body(relu, x_ref, p_ref, q_ref, s_ref, w_ref, b_ref, h_ref, y_ref):
    s = s_ref[...]
    u = (p_ref[0] + p_ref[1]) * s
    v = (q_ref[0] + q_ref[1]) * s
    a = w_ref[0] - w_ref[2]
    o = jnp.dot(x_ref[...], a, preferred_element_type=_f32)
    o = o - jnp.dot(u, w_ref[1], preferred_element_type=_f32)
    o = o - 2.0 * jnp.dot(v, w_ref[2], preferred_element_type=_f32)
    o = o + b_ref[...]
    if relu:
        o = jnp.maximum(o, 0.0)
    h_ref[...] = o
    y_ref[...] = o * s


def _layer(x, p, q, s, w, b, relu):
    return pl.pallas_call(
        functools.partial(_layer_body, relu),
        grid=(NACC // RBLK,),
        in_specs=[
            pl.BlockSpec((RBLK, FD), lambda i: (i, 0)),
            pl.BlockSpec((2, RBLK, FD), lambda i: (0, i, 0)),
            pl.BlockSpec((2, RBLK, FD), lambda i: (0, i, 0)),
            pl.BlockSpec((RBLK, 1), lambda i: (i, 0)),
            pl.BlockSpec((3, FD, FD), lambda i: (0, 0, 0)),
            pl.BlockSpec((1, FD), lambda i: (0, 0)),
        ],
        out_specs=(
            pl.BlockSpec((RBLK, FD), lambda i: (i, 0)),
            pl.BlockSpec((RBLK, FD), lambda i: (i, 0)),
        ),
        out_shape=(
            jax.ShapeDtypeStruct((NACC, FD), _f32),
            jax.ShapeDtypeStruct((NACC, FD), _f32),
        ),
    )(x, p, q, s, w, b)


def _final_body(h_ref, w_ref, b_ref, o_ref):
    logits = jnp.dot(h_ref[...], w_ref[...], preferred_element_type=_f32)
    logits = logits + b_ref[...]
    m = jnp.max(logits, axis=1, keepdims=True)
    z = logits - m
    lse = jnp.log(jnp.sum(jnp.exp(z), axis=1, keepdims=True))
    o_ref[...] = z - lse


def _final(h, w, b):
    return pl.pallas_call(
        _final_body,
        grid=(NACC // RBLK,),
        in_specs=[
            pl.BlockSpec((RBLK, FD), lambda i: (i, 0)),
            pl.BlockSpec((FD, FD), lambda i: (0, 0)),
            pl.BlockSpec((1, FD), lambda i: (0, 0)),
        ],
        out_specs=pl.BlockSpec((RBLK, FD), lambda i: (i, 0)),
        out_shape=jax.ShapeDtypeStruct((NACC, FD), _f32),
    )(h, w, b)


def kernel(x, edge_index, W1, b1, W2, b2, Wlin, blin):
    srcf, dstf, degp = _prep(edge_index)
    dis2, dgi2 = _dis(degp)
    dis = dis2.reshape(NACC, 1)
    dgi = dgi2.reshape(NACC, 1)

    x_pad = jnp.pad(x, ((0, NACC - N), (0, 0)))
    b1r = b1.reshape(1, FD)
    b2r = b2.reshape(1, FD)
    wl = jnp.pad(Wlin, ((0, 0), (0, FD - Wlin.shape[1])))
    bl = jnp.pad(blin, (0, FD - blin.shape[0]),
                 constant_values=-1e30).reshape(1, FD)

    y1 = _scale(x_pad, dis)
    p1 = _prop(y1, srcf, dstf)
    y2 = _mid(p1, dgi)
    p2 = _prop(y2, srcf, dstf)
    h1, y3 = _layer(x_pad, p1, p2, dis, W1, b1r, True)
    p3 = _prop(y3, srcf, dstf)
    y4 = _mid(p3, dgi)
    p4 = _prop(y4, srcf, dstf)
    h2, _ = _layer(h1, p3, p4, dis, W2, b2r, False)
    logp = _final(h2, wl, bl)
    return (logp[:N, :Wlin.shape[1]], edge_index)


# speedup vs baseline: 13.1260x; 1.5213x over previous
import functools

import jax
import jax.numpy as jnp
from jax import lax
from jax.experimental import pallas as pl
from jax.experimental.pallas import tpu as pltpu
from jax.experimental.pallas import tpu_sc as plsc

N = 10000
E = 320000
FD = 128
NW = 32
RPT = 640
NACC = 16 * RPT
CH = 128
NCH = 79
EPW = NCH * CH
EPAD = NW * EPW
RBLK = 640

_mesh = plsc.VectorSubcoreMesh(core_axis_name="c", subcore_axis_name="s")

_f32 = jnp.float32
_i32 = jnp.int32


@functools.partial(
    pl.kernel,
    mesh=_mesh,
    out_type=(
        jax.ShapeDtypeStruct((EPAD,), _i32),
        jax.ShapeDtypeStruct((EPAD,), _i32),
        jax.ShapeDtypeStruct((2, NACC), _f32),
    ),
    scratch_types=[
        pltpu.VMEM((CH,), _i32),
        pltpu.VMEM((CH,), _i32),
        pltpu.VMEM((CH,), _f32),
        pltpu.VMEM((RPT,), _f32),
        pltpu.VMEM_SHARED((NACC,), _f32),
        pltpu.SemaphoreType.DMA,
    ],
)
def _prep(ei, srcf, dstf, degp, rv, cv, ov, bv, dacc, sem):
    del sem
    cid = lax.axis_index("c")
    sid = lax.axis_index("s")
    wid = sid * 2 + cid

    def zb(i, carry):
        bv[pl.ds(i * 16, 16)] = jnp.zeros((16,), _f32)
        return carry

    lax.fori_loop(0, RPT // 16, zb, 0)
    pltpu.sync_copy(bv, dacc.at[pl.ds(sid * RPT, RPT)])
    plsc.subcore_barrier()

    def chunk(k, carry):
        base = wid * EPW + k * CH

        @pl.when(base < E)
        def _real():
            pltpu.sync_copy(ei.at[0, pl.ds(base, CH)], rv)
            pltpu.sync_copy(ei.at[1, pl.ds(base, CH)], cv)
            for j in range(CH // 16):
                sl = pl.ds(j * 16, 16)
                r = rv[sl]
                c = cv[sl]
                m = r == c
                rz = N + (r & 7)
                rv[sl] = jnp.where(m, rz, r)
                ov[sl] = jnp.where(m, jnp.zeros((16,), _f32),
                                   jnp.ones((16,), _f32))
            pltpu.sync_copy(rv, srcf.at[pl.ds(base, CH)])
            pltpu.sync_copy(cv, dstf.at[pl.ds(base, CH)])
            pltpu.sync_copy(ov, dacc.at[rv], add=True)

        @pl.when(base >= E)
        def _pad():
            ii = lax.iota(_i32, 16)
            for j in range(CH // 16):
                sl = pl.ds(j * 16, 16)
                rv[sl] = N + (ii & 7)
                cv[sl] = N + (ii & 15)
            pltpu.sync_copy(rv, srcf.at[pl.ds(base, CH)])
            pltpu.sync_copy(cv, dstf.at[pl.ds(base, CH)])

        return carry

    lax.fori_loop(0, NCH, chunk, 0)
    plsc.subcore_barrier()

    pltpu.sync_copy(dacc.at[pl.ds(sid * RPT, RPT)], bv)
    pltpu.sync_copy(bv, degp.at[cid, pl.ds(sid * RPT, RPT)])


@functools.partial(
    pl.kernel,
    mesh=_mesh,
    out_type=jax.ShapeDtypeStruct((2, NACC, FD), _f32),
    scratch_types=[
        pltpu.VMEM((CH,), _i32),
        pltpu.VMEM((CH,), _i32),
        pltpu.VMEM((CH, FD), _f32),
        pltpu.VMEM((CH,), _i32),
        pltpu.VMEM((CH,), _i32),
        pltpu.VMEM((CH, FD), _f32),
        pltpu.VMEM_SHARED((NACC, FD), _f32),
        pltpu.SemaphoreType.DMA,
        pltpu.SemaphoreType.DMA,
    ],
)
def _prop(tbl, src, dst, out, si0, di0, rows0, si1, di1, rows1, acc,
          sem0, sem1):
    cid = lax.axis_index("c")
    sid = lax.axis_index("s")
    wid = sid * 2 + cid
    ebase = wid * EPW

    def zrow(i, carry):
        for j in range(FD // 16):
            rows0[i, pl.ds(j * 16, 16)] = jnp.zeros((16,), _f32)
        return carry

    lax.fori_loop(0, CH, zrow, 0)
    for q in range(RPT // CH):
        pltpu.sync_copy(rows0, acc.at[pl.ds(sid * RPT + q * CH, CH)])
    plsc.subcore_barrier()

    def start(base, si, rows, sem, di):
        pltpu.sync_copy(src.at[pl.ds(base, CH)], si)
        pltpu.async_copy(tbl.at[si], rows, sem)
        pltpu.sync_copy(dst.at[pl.ds(base, CH)], di)

    def finish(si, rows, sem, di):
        pltpu.make_async_copy(tbl.at[si], rows, sem).wait()
        pltpu.sync_copy(rows, acc.at[di], add=True)

    start(ebase, si0, rows0, sem0, di0)

    def pair(i, carry):
        base = ebase + 2 * i * CH
        start(base + CH, si1, rows1, sem1, di1)
        finish(si0, rows0, sem0, di0)
        start(base + 2 * CH, si0, rows0, sem0, di0)
        finish(si1, rows1, sem1, di1)
        return carry

    lax.fori_loop(0, (NCH - 1) // 2, pair, 0)
    finish(si0, rows0, sem0, di0)
    plsc.subcore_barrier()

    for q in range(RPT // CH):
        r0 = sid * RPT + q * CH
        pltpu.sync_copy(acc.at[pl.ds(r0, CH)], rows0)
        pltpu.sync_copy(rows0, out.at[cid, pl.ds(r0, CH)])


def _dis_body(degp_ref, dis_ref, dgi_ref):
    deg = degp_ref[0] + degp_ref[1]
    good = deg > 0.0
    safe = jnp.where(good, deg, 1.0)
    r = lax.rsqrt(safe)
    dis_ref[...] = jnp.where(good, r, 0.0)
    dgi_ref[...] = jnp.where(good, 1.0 / safe, 0.0)


def _dis(degp):
    d2 = degp.reshape(2, NACC // 128, 128)
    return pl.pallas_call(
        _dis_body,
        out_shape=(
            jax.ShapeDtypeStruct((NACC // 128, 128), _f32),
            jax.ShapeDtypeStruct((NACC // 128, 128), _f32),
        ),
    )(d2)


def _scale_body(x_ref, s_ref, o_ref):
    o_ref[...] = x_ref[...] * s_ref[...]


def _scale(x, s):
    return pl.pallas_call(
        _scale_body,
        grid=(NACC // RBLK,),
        in_specs=[
            pl.BlockSpec((RBLK, FD), lambda i: (i, 0)),
            pl.BlockSpec((RBLK, 1), lambda i: (i, 0)),
        ],
        out_specs=pl.BlockSpec((RBLK, FD), lambda i: (i, 0)),
        out_shape=jax.ShapeDtypeStruct((NACC, FD), _f32),
    )(x, s)


def _mid_body(p_ref, g_ref, o_ref):
    o_ref[...] = (p_ref[0] + p_ref[1]) * (-g_ref[...])


def _mid(p, g):
    return pl.pallas_call(
        _mid_body,
        grid=(NACC // RBLK,),
        in_specs=[
            pl.BlockSpec((2, RBLK, FD), lambda i: (0, i, 0)),
            pl.BlockSpec((RBLK, 1), lambda i: (i, 0)),
        ],
        out_specs=pl.BlockSpec((RBLK, FD), lambda i: (i, 0)),
        out_shape=jax.ShapeDtypeStruct((NACC, FD), _f32),
    )(p, g)


def _layer_body(relu, x_ref, p_ref, q_ref, s_ref, w_ref, b_ref, h_ref, y_ref):
    s = s_ref[...]
    u = (p_ref[0] + p_ref[1]) * s
    v = (q_ref[0] + q_ref[1]) * s
    a = w_ref[0] - w_ref[2]
    o = jnp.dot(x_ref[...], a, preferred_element_type=_f32)
    o = o - jnp.dot(u, w_ref[1], preferred_element_type=_f32)
    o = o - 2.0 * jnp.dot(v, w_ref[2], preferred_element_type=_f32)
    o = o + b_ref[...]
    if relu:
        o = jnp.maximum(o, 0.0)
    h_ref[...] = o
    y_ref[...] = o * s


def _layer(x, p, q, s, w, b, relu):
    return pl.pallas_call(
        functools.partial(_layer_body, relu),
        grid=(NACC // RBLK,),
        in_specs=[
            pl.BlockSpec((RBLK, FD), lambda i: (i, 0)),
            pl.BlockSpec((2, RBLK, FD), lambda i: (0, i, 0)),
            pl.BlockSpec((2, RBLK, FD), lambda i: (0, i, 0)),
            pl.BlockSpec((RBLK, 1), lambda i: (i, 0)),
            pl.BlockSpec((3, FD, FD), lambda i: (0, 0, 0)),
            pl.BlockSpec((1, FD), lambda i: (0, 0)),
        ],
        out_specs=(
            pl.BlockSpec((RBLK, FD), lambda i: (i, 0)),
            pl.BlockSpec((RBLK, FD), lambda i: (i, 0)),
        ),
        out_shape=(
            jax.ShapeDtypeStruct((NACC, FD), _f32),
            jax.ShapeDtypeStruct((NACC, FD), _f32),
        ),
    )(x, p, q, s, w, b)


def _final_body(h_ref, w_ref, b_ref, o_ref):
    logits = jnp.dot(h_ref[...], w_ref[...], preferred_element_type=_f32)
    logits = logits + b_ref[...]
    m = jnp.max(logits, axis=1, keepdims=True)
    z = logits - m
    lse = jnp.log(jnp.sum(jnp.exp(z), axis=1, keepdims=True))
    o_ref[...] = z - lse


def _final(h, w, b):
    return pl.pallas_call(
        _final_body,
        grid=(NACC // RBLK,),
        in_specs=[
            pl.BlockSpec((RBLK, FD), lambda i: (i, 0)),
            pl.BlockSpec((FD, FD), lambda i: (0, 0)),
            pl.BlockSpec((1, FD), lambda i: (0, 0)),
        ],
        out_specs=pl.BlockSpec((RBLK, FD), lambda i: (i, 0)),
        out_shape=jax.ShapeDtypeStruct((NACC, FD), _f32),
    )(h, w, b)


def kernel(x, edge_index, W1, b1, W2, b2, Wlin, blin):
    srcf, dstf, degp = _prep(edge_index)
    dis2, dgi2 = _dis(degp)
    dis = dis2.reshape(NACC, 1)
    dgi = dgi2.reshape(NACC, 1)

    x_pad = jnp.pad(x, ((0, NACC - N), (0, 0)))
    b1r = b1.reshape(1, FD)
    b2r = b2.reshape(1, FD)
    wl = jnp.pad(Wlin, ((0, 0), (0, FD - Wlin.shape[1])))
    bl = jnp.pad(blin, (0, FD - blin.shape[0]),
                 constant_values=-1e30).reshape(1, FD)

    y1 = _scale(x_pad, dis)
    p1 = _prop(y1, srcf, dstf)
    y2 = _mid(p1, dgi)
    p2 = _prop(y2, srcf, dstf)
    h1, y3 = _layer(x_pad, p1, p2, dis, W1, b1r, True)
    p3 = _prop(y3, srcf, dstf)
    y4 = _mid(p3, dgi)
    p4 = _prop(y4, srcf, dstf)
    h2, _ = _layer(h1, p3, p4, dis, W2, b2r, False)
    logp = _final(h2, wl, bl)
    return (logp[:N, :Wlin.shape[1]], edge_index)


# speedup vs baseline: 13.1373x; 1.0009x over previous
import functools

import jax
import jax.numpy as jnp
from jax import lax
from jax.experimental import pallas as pl
from jax.experimental.pallas import tpu as pltpu
from jax.experimental.pallas import tpu_sc as plsc

N = 10000
E = 320000
FD = 128
NW = 32
RPT = 640
NACC = 16 * RPT
CH = 128
NCH = 79
EPW = NCH * CH
EPAD = NW * EPW
RBLK = 640

_mesh = plsc.VectorSubcoreMesh(core_axis_name="c", subcore_axis_name="s")

_f32 = jnp.float32
_i32 = jnp.int32


@functools.partial(
    pl.kernel,
    mesh=_mesh,
    out_type=(
        jax.ShapeDtypeStruct((EPAD,), _i32),
        jax.ShapeDtypeStruct((EPAD,), _i32),
        jax.ShapeDtypeStruct((2, NACC), _f32),
    ),
    scratch_types=[
        pltpu.VMEM((CH,), _i32),
        pltpu.VMEM((CH,), _i32),
        pltpu.VMEM((CH,), _f32),
        pltpu.VMEM((RPT,), _f32),
        pltpu.VMEM_SHARED((NACC,), _f32),
        pltpu.SemaphoreType.DMA,
    ],
)
def _prep(ei, srcf, dstf, degp, rv, cv, ov, bv, dacc, sem):
    del sem
    cid = lax.axis_index("c")
    sid = lax.axis_index("s")
    wid = sid * 2 + cid

    def zb(i, carry):
        bv[pl.ds(i * 16, 16)] = jnp.zeros((16,), _f32)
        return carry

    lax.fori_loop(0, RPT // 16, zb, 0)
    pltpu.sync_copy(bv, dacc.at[pl.ds(sid * RPT, RPT)])
    plsc.subcore_barrier()

    def chunk(k, carry):
        base = wid * EPW + k * CH

        @pl.when(base < E)
        def _real():
            pltpu.sync_copy(ei.at[0, pl.ds(base, CH)], rv)
            pltpu.sync_copy(ei.at[1, pl.ds(base, CH)], cv)
            for j in range(CH // 16):
                sl = pl.ds(j * 16, 16)
                r = rv[sl]
                c = cv[sl]
                m = r == c
                rz = N + (r & 7)
                rv[sl] = jnp.where(m, rz, r)
                ov[sl] = jnp.where(m, jnp.zeros((16,), _f32),
                                   jnp.ones((16,), _f32))
            pltpu.sync_copy(rv, srcf.at[pl.ds(base, CH)])
            pltpu.sync_copy(cv, dstf.at[pl.ds(base, CH)])
            pltpu.sync_copy(ov, dacc.at[rv], add=True)

        @pl.when(base >= E)
        def _pad():
            ii = lax.iota(_i32, 16)
            for j in range(CH // 16):
                sl = pl.ds(j * 16, 16)
                rv[sl] = N + (ii & 7)
                cv[sl] = N + (ii & 15)
            pltpu.sync_copy(rv, srcf.at[pl.ds(base, CH)])
            pltpu.sync_copy(cv, dstf.at[pl.ds(base, CH)])

        return carry

    lax.fori_loop(0, NCH, chunk, 0)
    plsc.subcore_barrier()

    pltpu.sync_copy(dacc.at[pl.ds(sid * RPT, RPT)],
                    degp.at[cid, pl.ds(sid * RPT, RPT)])


@functools.partial(
    pl.kernel,
    mesh=_mesh,
    out_type=jax.ShapeDtypeStruct((2, NACC, FD), _f32),
    scratch_types=[
        pltpu.VMEM((CH,), _i32),
        pltpu.VMEM((CH,), _i32),
        pltpu.VMEM((CH, FD), _f32),
        pltpu.VMEM((CH,), _i32),
        pltpu.VMEM((CH,), _i32),
        pltpu.VMEM((CH, FD), _f32),
        pltpu.VMEM_SHARED((NACC, FD), _f32),
        pltpu.SemaphoreType.DMA,
        pltpu.SemaphoreType.DMA,
    ],
)
def _prop(tbl, src, dst, out, si0, di0, rows0, si1, di1, rows1, acc,
          sem0, sem1):
    cid = lax.axis_index("c")
    sid = lax.axis_index("s")
    wid = sid * 2 + cid
    ebase = wid * EPW

    def zrow(i, carry):
        for j in range(FD // 16):
            rows0[i, pl.ds(j * 16, 16)] = jnp.zeros((16,), _f32)
        return carry

    lax.fori_loop(0, CH, zrow, 0)
    for q in range(RPT // CH):
        pltpu.sync_copy(rows0, acc.at[pl.ds(sid * RPT + q * CH, CH)])
    plsc.subcore_barrier()

    def start(base, si, rows, sem, di):
        pltpu.sync_copy(src.at[pl.ds(base, CH)], si)
        pltpu.async_copy(tbl.at[si], rows, sem)
        pltpu.sync_copy(dst.at[pl.ds(base, CH)], di)

    def finish(si, rows, sem, di):
        pltpu.make_async_copy(tbl.at[si], rows, sem).wait()
        pltpu.sync_copy(rows, acc.at[di], add=True)

    start(ebase, si0, rows0, sem0, di0)

    def pair(i, carry):
        base = ebase + 2 * i * CH
        start(base + CH, si1, rows1, sem1, di1)
        finish(si0, rows0, sem0, di0)
        start(base + 2 * CH, si0, rows0, sem0, di0)
        finish(si1, rows1, sem1, di1)
        return carry

    lax.fori_loop(0, (NCH - 1) // 2, pair, 0)
    finish(si0, rows0, sem0, di0)
    plsc.subcore_barrier()

    r0_ = sid * RPT
    pltpu.sync_copy(acc.at[pl.ds(r0_, RPT)], out.at[cid, pl.ds(r0_, RPT)])


def _dis_body(degp_ref, dis_ref, dgi_ref):
    deg = degp_ref[0] + degp_ref[1]
    good = deg > 0.0
    safe = jnp.where(good, deg, 1.0)
    r = lax.rsqrt(safe)
    dis_ref[...] = jnp.where(good, r, 0.0)
    dgi_ref[...] = jnp.where(good, 1.0 / safe, 0.0)


def _dis(degp):
    d2 = degp.reshape(2, NACC // 128, 128)
    return pl.pallas_call(
        _dis_body,
        out_shape=(
            jax.ShapeDtypeStruct((NACC // 128, 128), _f32),
            jax.ShapeDtypeStruct((NACC // 128, 128), _f32),
        ),
    )(d2)


def _scale_body(x_ref, s_ref, o_ref):
    o_ref[...] = x_ref[...] * s_ref[...]


def _scale(x, s):
    return pl.pallas_call(
        _scale_body,
        grid=(NACC // RBLK,),
        in_specs=[
            pl.BlockSpec((RBLK, FD), lambda i: (i, 0)),
            pl.BlockSpec((RBLK, 1), lambda i: (i, 0)),
        ],
        out_specs=pl.BlockSpec((RBLK, FD), lambda i: (i, 0)),
        out_shape=jax.ShapeDtypeStruct((NACC, FD), _f32),
    )(x, s)


def _mid_body(p_ref, g_ref, o_ref):
    o_ref[...] = (p_ref[0] + p_ref[1]) * (-g_ref[...])


def _mid(p, g):
    return pl.pallas_call(
        _mid_body,
        grid=(NACC // RBLK,),
        in_specs=[
            pl.BlockSpec((2, RBLK, FD), lambda i: (0, i, 0)),
            pl.BlockSpec((RBLK, 1), lambda i: (i, 0)),
        ],
        out_specs=pl.BlockSpec((RBLK, FD), lambda i: (i, 0)),
        out_shape=jax.ShapeDtypeStruct((NACC, FD), _f32),
    )(p, g)


def _layer_body(relu, x_ref, p_ref, q_ref, s_ref, w_ref, b_ref, h_ref, y_ref):
    s = s_ref[...]
    u = (p_ref[0] + p_ref[1]) * s
    v = (q_ref[0] + q_ref[1]) * s
    a = w_ref[0] - w_ref[2]
    o = jnp.dot(x_ref[...], a, preferred_element_type=_f32)
    o = o - jnp.dot(u, w_ref[1], preferred_element_type=_f32)
    o = o - 2.0 * jnp.dot(v, w_ref[2], preferred_element_type=_f32)
    o = o + b_ref[...]
    if relu:
        o = jnp.maximum(o, 0.0)
    h_ref[...] = o
    y_ref[...] = o * s


def _layer(x, p, q, s, w, b, relu):
    return pl.pallas_call(
        functools.partial(_layer_body, relu),
        grid=(NACC // RBLK,),
        in_specs=[
            pl.BlockSpec((RBLK, FD), lambda i: (i, 0)),
            pl.BlockSpec((2, RBLK, FD), lambda i: (0, i, 0)),
            pl.BlockSpec((2, RBLK, FD), lambda i: (0, i, 0)),
            pl.BlockSpec((RBLK, 1), lambda i: (i, 0)),
            pl.BlockSpec((3, FD, FD), lambda i: (0, 0, 0)),
            pl.BlockSpec((1, FD), lambda i: (0, 0)),
        ],
        out_specs=(
            pl.BlockSpec((RBLK, FD), lambda i: (i, 0)),
            pl.BlockSpec((RBLK, FD), lambda i: (i, 0)),
        ),
        out_shape=(
            jax.ShapeDtypeStruct((NACC, FD), _f32),
            jax.ShapeDtypeStruct((NACC, FD), _f32),
        ),
    )(x, p, q, s, w, b)


def _final_body(h_ref, w_ref, b_ref, o_ref):
    logits = jnp.dot(h_ref[...], w_ref[...], preferred_element_type=_f32)
    logits = logits + b_ref[...]
    m = jnp.max(logits, axis=1, keepdims=True)
    z = logits - m
    lse = jnp.log(jnp.sum(jnp.exp(z), axis=1, keepdims=True))
    o_ref[...] = z - lse


def _final(h, w, b):
    return pl.pallas_call(
        _final_body,
        grid=(NACC // RBLK,),
        in_specs=[
            pl.BlockSpec((RBLK, FD), lambda i: (i, 0)),
            pl.BlockSpec((FD, FD), lambda i: (0, 0)),
            pl.BlockSpec((1, FD), lambda i: (0, 0)),
        ],
        out_specs=pl.BlockSpec((RBLK, FD), lambda i: (i, 0)),
        out_shape=jax.ShapeDtypeStruct((NACC, FD), _f32),
    )(h, w, b)


def kernel(x, edge_index, W1, b1, W2, b2, Wlin, blin):
    srcf, dstf, degp = _prep(edge_index)
    dis2, dgi2 = _dis(degp)
    dis = dis2.reshape(NACC, 1)
    dgi = dgi2.reshape(NACC, 1)

    x_pad = jnp.pad(x, ((0, NACC - N), (0, 0)))
    b1r = b1.reshape(1, FD)
    b2r = b2.reshape(1, FD)
    wl = jnp.pad(Wlin, ((0, 0), (0, FD - Wlin.shape[1])))
    bl = jnp.pad(blin, (0, FD - blin.shape[0]),
                 constant_values=-1e30).reshape(1, FD)

    y1 = _scale(x_pad, dis)
    p1 = _prop(y1, srcf, dstf)
    y2 = _mid(p1, dgi)
    p2 = _prop(y2, srcf, dstf)
    h1, y3 = _layer(x_pad, p1, p2, dis, W1, b1r, True)
    p3 = _prop(y3, srcf, dstf)
    y4 = _mid(p3, dgi)
    p4 = _prop(y4, srcf, dstf)
    h2, _ = _layer(h1, p3, p4, dis, W2, b2r, False)
    logp = _final(h2, wl, bl)
    return (logp[:N, :Wlin.shape[1]], edge_index)
